# baseline (speedup 1.0000x reference)
import jax
import jax.numpy as jnp
from jax.experimental import pallas as pl
from jax.experimental.pallas import tpu as pltpu

_B = 4096
_C = 9605
_CLIP = 0.05
_EPS = 1e-08
_ALPHA = 0.5

_BC = 256
_BRC = 2408


def _elem_loss(x, y):
    t = jnp.tanh(x * 0.5)
    v = jnp.where(y > 0.5,
                  jnp.maximum(0.5 + 0.5 * t, _EPS),
                  jnp.minimum(0.55 - 0.5 * t, 1.0))
    return jnp.log(v)


def _loss_body(x_ref, y_ref, out_ref):
    i = pl.program_id(0)
    j = pl.program_id(1)
    loss = _elem_loss(x_ref[...], y_ref[...])
    row = jax.lax.broadcasted_iota(jnp.int32, loss.shape, 0)
    loss = jnp.where(row < _C - j * _BRC, loss, 0.0)
    blk = jnp.sum(loss)

    @pl.when(j == 0)
    def _corr():
        x12 = x_ref[0:12, :]
        y12 = y_ref[0:12, :]
        l12 = _elem_loss(x12, y12)
        s_r = jnp.sum(y12[0:5, :], axis=0)
        s_d = jnp.sum(y12[5:9, :], axis=0)
        s_c = jnp.sum(y12[9:12, :], axis=0)
        L_r = jnp.sum(l12[0:5, :], axis=0)
        L_d = jnp.sum(l12[5:9, :], axis=0)
        L_c = jnp.sum(l12[9:12, :], axis=0)
        any_active = (s_r > 0) | (s_d > 0) | (s_c > 0)
        inactive_loss = (jnp.where(s_r == 0, L_r, 0.0)
                         + jnp.where(s_d == 0, L_d, 0.0)
                         + jnp.where(s_c == 0, L_c, 0.0))
        corr = jnp.sum(jnp.where(any_active, inactive_loss, 0.0))

        @pl.when((i == 0) & (j == 0))
        def _init():
            out_ref[0, 0] = jnp.float32(0.0)

        out_ref[0, 0] += -(1.0 - _ALPHA) * corr

    out_ref[0, 0] += blk


@jax.jit
def kernel(x, y):
    xt = x.T
    yt = y.T
    out = pl.pallas_call(
        _loss_body,
        grid=(_B // _BC, pl.cdiv(_C, _BRC)),
        in_specs=[
            pl.BlockSpec((_BRC, _BC), lambda i, j: (j, i)),
            pl.BlockSpec((_BRC, _BC), lambda i, j: (j, i)),
        ],
        out_specs=pl.BlockSpec((1, 1), lambda i, j: (0, 0),
                               memory_space=pltpu.SMEM),
        out_shape=jax.ShapeDtypeStruct((1, 1), jnp.float32),
    )(xt, yt)
    return -out[0, 0]


# speedup vs baseline: 1.3073x; 1.3073x over previous
import jax
import jax.numpy as jnp
from jax.experimental import pallas as pl
from jax.experimental.pallas import tpu as pltpu

_B = 4096
_C = 9605
_CLIP = 0.05
_EPS = 1e-08
_ALPHA = 0.5

_BC = 256


def _elem_loss(x, y):
    t = jnp.tanh(x * 0.5)
    v = jnp.where(y > 0.5,
                  jnp.maximum(0.5 + 0.5 * t, _EPS),
                  jnp.minimum(0.55 - 0.5 * t, 1.0))
    return jnp.log(v)


def _loss_body(x_ref, y_ref, out_ref):
    i = pl.program_id(0)
    loss = _elem_loss(x_ref[...], y_ref[...])
    total = jnp.sum(loss)

    x12 = x_ref[0:12, :]
    y12 = y_ref[0:12, :]
    l12 = _elem_loss(x12, y12)
    s_r = jnp.sum(y12[0:5, :], axis=0)
    s_d = jnp.sum(y12[5:9, :], axis=0)
    s_c = jnp.sum(y12[9:12, :], axis=0)
    L_r = jnp.sum(l12[0:5, :], axis=0)
    L_d = jnp.sum(l12[5:9, :], axis=0)
    L_c = jnp.sum(l12[9:12, :], axis=0)
    any_active = (s_r > 0) | (s_d > 0) | (s_c > 0)
    inactive_loss = (jnp.where(s_r == 0, L_r, 0.0)
                     + jnp.where(s_d == 0, L_d, 0.0)
                     + jnp.where(s_c == 0, L_c, 0.0))
    corr = jnp.sum(jnp.where(any_active, inactive_loss, 0.0))

    blk = total - (1.0 - _ALPHA) * corr

    @pl.when(i == 0)
    def _init():
        out_ref[0, 0] = jnp.float32(0.0)

    out_ref[0, 0] += blk


@jax.jit
def kernel(x, y):
    xt = x.T
    yt = y.T
    out = pl.pallas_call(
        _loss_body,
        grid=(_B // _BC,),
        in_specs=[
            pl.BlockSpec((_C, _BC), lambda i: (0, i)),
            pl.BlockSpec((_C, _BC), lambda i: (0, i)),
        ],
        out_specs=pl.BlockSpec((1, 1), lambda i: (0, 0),
                               memory_space=pltpu.SMEM),
        out_shape=jax.ShapeDtypeStruct((1, 1), jnp.float32),
    )(xt, yt)
    return -out[0, 0]
